# baseline (speedup 1.0000x reference)
import functools

import jax
import jax.numpy as jnp
from jax import lax
from jax.experimental import pallas as pl
from jax.experimental.pallas import tpu as pltpu
from jax.experimental.pallas import tpu_sc as plsc

N = 10000
E = 320000
D = 128
K = 5
C = 64

BR = 400
NB = N // BR
NC, NS = 2, 16
NW = NC * NS
B = 80
NCHUNK = E // B
NCW = NCHUNK // NW
NIH = NCW // 4
NROWS = N
RPT = 624


def _softmax_row(hop_ref):
    hrow = hop_ref[...]
    m = jnp.max(hrow)
    e = jnp.exp(hrow - m)
    return e / jnp.sum(e)


def _mlp_scale_body(x_ref, w_ref, b_ref, hop_ref, h_ref, s_ref):
    h = jnp.maximum(
        jnp.dot(x_ref[...], w_ref[...], preferred_element_type=jnp.float32)
        + b_ref[...], 0.0)
    h_ref[...] = h
    w = _softmax_row(hop_ref)
    for kk in range(K):
        s_ref[kk] = h * w[0, kk]


_mlp_scale = pl.pallas_call(
    _mlp_scale_body,
    grid=(NB,),
    in_specs=[pl.BlockSpec((BR, D), lambda i: (i, 0)),
              pl.BlockSpec((D, D), lambda i: (0, 0)),
              pl.BlockSpec((1, D), lambda i: (0, 0)),
              pl.BlockSpec((1, K), lambda i: (0, 0))],
    out_specs=[pl.BlockSpec((BR, D), lambda i: (i, 0)),
               pl.BlockSpec((K, BR, D), lambda i: (0, i, 0))],
    out_shape=[jax.ShapeDtypeStruct((N, D), jnp.float32),
               jax.ShapeDtypeStruct((K, N, D), jnp.float32)],
)


def _combine_scale_body(h_ref, p_ref, w_ref, b_ref, hop_ref,
                        h1_ref, s_ref):
    s = h_ref[...] + p_ref[0] + p_ref[1]
    h1 = jnp.maximum(
        jnp.dot(s, w_ref[...], preferred_element_type=jnp.float32)
        + b_ref[...], 0.0)
    h1_ref[...] = h1
    w = _softmax_row(hop_ref)
    for kk in range(K):
        s_ref[kk] = h1 * w[0, kk]


_combine_scale = pl.pallas_call(
    _combine_scale_body,
    grid=(NB,),
    in_specs=[pl.BlockSpec((BR, D), lambda i: (i, 0)),
              pl.BlockSpec((NC, BR, D), lambda i: (0, i, 0)),
              pl.BlockSpec((D, D), lambda i: (0, 0)),
              pl.BlockSpec((1, D), lambda i: (0, 0)),
              pl.BlockSpec((1, K), lambda i: (0, 0))],
    out_specs=[pl.BlockSpec((BR, D), lambda i: (i, 0)),
               pl.BlockSpec((K, BR, D), lambda i: (0, i, 0))],
    out_shape=[jax.ShapeDtypeStruct((N, D), jnp.float32),
               jax.ShapeDtypeStruct((K, N, D), jnp.float32)],
)


def _combine_head_body(h_ref, p_ref, w_ref, b_ref,
                       w1_ref, b1_ref, w2_ref, b2_ref, o_ref):
    s = h_ref[...] + p_ref[0] + p_ref[1]
    h2 = jnp.maximum(
        jnp.dot(s, w_ref[...], preferred_element_type=jnp.float32)
        + b_ref[...], 0.0)
    t = jnp.maximum(
        jnp.dot(h2, w1_ref[...], preferred_element_type=jnp.float32)
        + b1_ref[...], 0.0)
    o_ref[...] = (jnp.dot(t, w2_ref[...], preferred_element_type=jnp.float32)
                  + b2_ref[...])


_combine_head = pl.pallas_call(
    _combine_head_body,
    grid=(NB,),
    in_specs=[pl.BlockSpec((BR, D), lambda i: (i, 0)),
              pl.BlockSpec((NC, BR, D), lambda i: (0, i, 0)),
              pl.BlockSpec((D, D), lambda i: (0, 0)),
              pl.BlockSpec((1, D), lambda i: (0, 0)),
              pl.BlockSpec((D, D), lambda i: (0, 0)),
              pl.BlockSpec((1, D), lambda i: (0, 0)),
              pl.BlockSpec((D, C), lambda i: (0, 0)),
              pl.BlockSpec((1, C), lambda i: (0, 0))],
    out_specs=pl.BlockSpec((BR, C), lambda i: (i, 0)),
    out_shape=jax.ShapeDtypeStruct((N, C), jnp.float32),
)


def _gidx_body(src_ref, ew_ref, o_ref):
    o_ref[...] = ew_ref[...] * N + src_ref[...]


_gidx = pl.pallas_call(
    _gidx_body,
    out_shape=jax.ShapeDtypeStruct((NCHUNK, B), jnp.int32),
)


_mesh = plsc.VectorSubcoreMesh(core_axis_name="c", subcore_axis_name="s")


@functools.partial(
    pl.kernel,
    out_type=jax.ShapeDtypeStruct((NC, N, D), jnp.float32),
    mesh=_mesh,
    scratch_types=[
        pltpu.VMEM((B,), jnp.int32),
        pltpu.VMEM((B,), jnp.int32),
        pltpu.VMEM((B,), jnp.int32),
        pltpu.VMEM((B,), jnp.int32),
        pltpu.VMEM((B,), jnp.int32),
        pltpu.VMEM((B,), jnp.int32),
        pltpu.VMEM((B,), jnp.int32),
        pltpu.VMEM((B,), jnp.int32),
        pltpu.VMEM((B, D), jnp.float32),
        pltpu.VMEM((B, D), jnp.float32),
        pltpu.VMEM((B, D), jnp.float32),
        pltpu.VMEM((B, D), jnp.float32),
        pltpu.VMEM_SHARED((NROWS, D), jnp.float32),
        pltpu.SemaphoreType.DMA,
        pltpu.SemaphoreType.DMA,
        pltpu.SemaphoreType.DMA,
        pltpu.SemaphoreType.DMA,
    ],
)
def _sc_agg(scaled_hbm, gidx_hbm, dst_hbm, out_hbm,
            gi0_v, di0_v, gi1_v, di1_v, gi2_v, di2_v, gi3_v, di3_v,
            rows0_v, rows1_v, rows2_v, rows3_v, acc_sh,
            sem0, sem1, sem2, sem3):
    cid = lax.axis_index("c")
    sid = lax.axis_index("s")
    wid = cid * NS + sid

    zv = jnp.zeros((16,), jnp.float32)

    def _zb(i, carry):
        rows0_v[i // 8, pl.ds((i % 8) * 16, 16)] = zv
        return carry

    lax.fori_loop(0, B * 8, _zb, 0)
    r0 = sid * RPT
    nz = RPT // B
    for j in range(nz):
        pltpu.sync_copy(rows0_v, acc_sh.at[pl.ds(r0 + j * B, B)])
    rem = RPT - nz * B
    pltpu.sync_copy(rows0_v.at[pl.ds(0, rem)],
                    acc_sh.at[pl.ds(r0 + nz * B, rem)])

    @pl.when(sid == 0)
    def _zero_tail():
        pltpu.sync_copy(rows0_v.at[pl.ds(0, 16)],
                        acc_sh.at[pl.ds(NS * RPT, 16)])

    plsc.subcore_barrier()

    base = NCW * wid * B
    slots = ((gi0_v, di0_v, rows0_v, sem0), (gi1_v, di1_v, rows1_v, sem1),
             (gi2_v, di2_v, rows2_v, sem2), (gi3_v, di3_v, rows3_v, sem3))

    def _quad(i, carry):
        off0 = base + (4 * i) * B
        ixs = []
        for j, (gi_v, di_v, rows_v, sem) in enumerate(slots):
            off = off0 + j * B
            ixs.append((pltpu.async_copy(gidx_hbm.at[pl.ds(off, B)], gi_v,
                                         sem),
                        pltpu.async_copy(dst_hbm.at[pl.ds(off, B)], di_v,
                                         sem)))
        gs = []
        for (ia, ib), (gi_v, di_v, rows_v, sem) in zip(ixs, slots):
            ia.wait()
            ib.wait()
            gs.append(pltpu.async_copy(scaled_hbm.at[gi_v], rows_v, sem))
        for g, (gi_v, di_v, rows_v, sem) in zip(gs, slots):
            g.wait()
            pltpu.sync_copy(rows_v, acc_sh.at[di_v], add=True)
        return carry

    lax.fori_loop(0, NIH, _quad, 0)

    off = base + (NCW - 1) * B
    pltpu.sync_copy(gidx_hbm.at[pl.ds(off, B)], gi0_v)
    pltpu.sync_copy(dst_hbm.at[pl.ds(off, B)], di0_v)
    pltpu.async_copy(scaled_hbm.at[gi0_v], rows0_v, sem0).wait()
    pltpu.sync_copy(rows0_v, acc_sh.at[di0_v], add=True)

    plsc.subcore_barrier()
    pltpu.sync_copy(acc_sh.at[pl.ds(r0, RPT)],
                    out_hbm.at[cid, pl.ds(r0, RPT)])

    @pl.when(sid == 0)
    def _flush_tail():
        pltpu.sync_copy(acc_sh.at[pl.ds(NS * RPT, 16)],
                        out_hbm.at[cid, pl.ds(NS * RPT, 16)])


def kernel(x, edge_index, edge_weights, W0, b0, hop1, W1, b1,
           hop2, W2, b2, Wh1, bh1, Wh2, bh2):
    src = edge_index[0]
    dst = edge_index[1]
    gidx = _gidx(src.reshape(NCHUNK, B),
                 edge_weights.reshape(NCHUNK, B)).reshape(E)

    h0, s1 = _mlp_scale(x, W0, b0.reshape(1, D), hop1.reshape(1, K))
    p1 = _sc_agg(s1.reshape(K * N, D), gidx, dst)
    h1, s2 = _combine_scale(h0, p1, W1, b1.reshape(1, D),
                            hop2.reshape(1, K))
    p2 = _sc_agg(s2.reshape(K * N, D), gidx, dst)
    out = _combine_head(h1, p2, W2, b2.reshape(1, D),
                        Wh1, bh1.reshape(1, D), Wh2, bh2.reshape(1, C))
    return out


# speedup vs baseline: 1.0243x; 1.0243x over previous
import functools

import jax
import jax.numpy as jnp
from jax import lax
from jax.experimental import pallas as pl
from jax.experimental.pallas import tpu as pltpu
from jax.experimental.pallas import tpu_sc as plsc

N = 10000
E = 320000
D = 128
K = 5
C = 64

BR = 400
NB = N // BR
NC, NS = 2, 16
NW = NC * NS
B = 128
NCHUNK = E // B
NCW = NCHUNK // NW
NCX = NCHUNK % NW
NIH = NCW // 3
NROWS = N
RPT = 624


def _softmax_row(hop_ref):
    hrow = hop_ref[...]
    m = jnp.max(hrow)
    e = jnp.exp(hrow - m)
    return e / jnp.sum(e)


def _mlp_scale_body(x_ref, w_ref, b_ref, hop_ref, h_ref, s_ref):
    h = jnp.maximum(
        jnp.dot(x_ref[...], w_ref[...], preferred_element_type=jnp.float32)
        + b_ref[...], 0.0)
    h_ref[...] = h
    w = _softmax_row(hop_ref)
    for kk in range(K):
        s_ref[kk] = h * w[0, kk]


_mlp_scale = pl.pallas_call(
    _mlp_scale_body,
    grid=(NB,),
    in_specs=[pl.BlockSpec((BR, D), lambda i: (i, 0)),
              pl.BlockSpec((D, D), lambda i: (0, 0)),
              pl.BlockSpec((1, D), lambda i: (0, 0)),
              pl.BlockSpec((1, K), lambda i: (0, 0))],
    out_specs=[pl.BlockSpec((BR, D), lambda i: (i, 0)),
               pl.BlockSpec((K, BR, D), lambda i: (0, i, 0))],
    out_shape=[jax.ShapeDtypeStruct((N, D), jnp.float32),
               jax.ShapeDtypeStruct((K, N, D), jnp.float32)],
)


def _combine_scale_body(h_ref, p_ref, w_ref, b_ref, hop_ref,
                        h1_ref, s_ref):
    s = h_ref[...] + p_ref[0] + p_ref[1]
    h1 = jnp.maximum(
        jnp.dot(s, w_ref[...], preferred_element_type=jnp.float32)
        + b_ref[...], 0.0)
    h1_ref[...] = h1
    w = _softmax_row(hop_ref)
    for kk in range(K):
        s_ref[kk] = h1 * w[0, kk]


_combine_scale = pl.pallas_call(
    _combine_scale_body,
    grid=(NB,),
    in_specs=[pl.BlockSpec((BR, D), lambda i: (i, 0)),
              pl.BlockSpec((NC, BR, D), lambda i: (0, i, 0)),
              pl.BlockSpec((D, D), lambda i: (0, 0)),
              pl.BlockSpec((1, D), lambda i: (0, 0)),
              pl.BlockSpec((1, K), lambda i: (0, 0))],
    out_specs=[pl.BlockSpec((BR, D), lambda i: (i, 0)),
               pl.BlockSpec((K, BR, D), lambda i: (0, i, 0))],
    out_shape=[jax.ShapeDtypeStruct((N, D), jnp.float32),
               jax.ShapeDtypeStruct((K, N, D), jnp.float32)],
)


def _combine_head_body(h_ref, p_ref, w_ref, b_ref,
                       w1_ref, b1_ref, w2_ref, b2_ref, o_ref):
    s = h_ref[...] + p_ref[0] + p_ref[1]
    h2 = jnp.maximum(
        jnp.dot(s, w_ref[...], preferred_element_type=jnp.float32)
        + b_ref[...], 0.0)
    t = jnp.maximum(
        jnp.dot(h2, w1_ref[...], preferred_element_type=jnp.float32)
        + b1_ref[...], 0.0)
    o_ref[...] = (jnp.dot(t, w2_ref[...], preferred_element_type=jnp.float32)
                  + b2_ref[...])


_combine_head = pl.pallas_call(
    _combine_head_body,
    grid=(NB,),
    in_specs=[pl.BlockSpec((BR, D), lambda i: (i, 0)),
              pl.BlockSpec((NC, BR, D), lambda i: (0, i, 0)),
              pl.BlockSpec((D, D), lambda i: (0, 0)),
              pl.BlockSpec((1, D), lambda i: (0, 0)),
              pl.BlockSpec((D, D), lambda i: (0, 0)),
              pl.BlockSpec((1, D), lambda i: (0, 0)),
              pl.BlockSpec((D, C), lambda i: (0, 0)),
              pl.BlockSpec((1, C), lambda i: (0, 0))],
    out_specs=pl.BlockSpec((BR, C), lambda i: (i, 0)),
    out_shape=jax.ShapeDtypeStruct((N, C), jnp.float32),
)


def _gidx_body(src_ref, ew_ref, o_ref):
    o_ref[...] = ew_ref[...] * N + src_ref[...]


_gidx = pl.pallas_call(
    _gidx_body,
    out_shape=jax.ShapeDtypeStruct((NCHUNK, B), jnp.int32),
)


_mesh = plsc.VectorSubcoreMesh(core_axis_name="c", subcore_axis_name="s")


@functools.partial(
    pl.kernel,
    out_type=jax.ShapeDtypeStruct((NC, N, D), jnp.float32),
    mesh=_mesh,
    scratch_types=[
        pltpu.VMEM((B,), jnp.int32),
        pltpu.VMEM((B,), jnp.int32),
        pltpu.VMEM((B,), jnp.int32),
        pltpu.VMEM((B,), jnp.int32),
        pltpu.VMEM((B,), jnp.int32),
        pltpu.VMEM((B,), jnp.int32),
        pltpu.VMEM((B, D), jnp.float32),
        pltpu.VMEM((B, D), jnp.float32),
        pltpu.VMEM((B, D), jnp.float32),
        pltpu.VMEM_SHARED((NROWS, D), jnp.float32),
        pltpu.SemaphoreType.DMA,
        pltpu.SemaphoreType.DMA,
        pltpu.SemaphoreType.DMA,
    ],
)
def _sc_agg(scaled_hbm, gidx_hbm, dst_hbm, out_hbm,
            gi0_v, di0_v, gi1_v, di1_v, gi2_v, di2_v,
            rows0_v, rows1_v, rows2_v, acc_sh, sem0, sem1, sem2):
    cid = lax.axis_index("c")
    sid = lax.axis_index("s")
    wid = cid * NS + sid

    zv = jnp.zeros((16,), jnp.float32)

    def _zb(i, carry):
        rows0_v[i // 8, pl.ds((i % 8) * 16, 16)] = zv
        return carry

    lax.fori_loop(0, B * 8, _zb, 0)
    r0 = sid * RPT
    nz = RPT // B
    for j in range(nz):
        pltpu.sync_copy(rows0_v, acc_sh.at[pl.ds(r0 + j * B, B)])
    rem = RPT - nz * B
    pltpu.sync_copy(rows0_v.at[pl.ds(0, rem)],
                    acc_sh.at[pl.ds(r0 + nz * B, rem)])

    @pl.when(sid == 0)
    def _zero_tail():
        pltpu.sync_copy(rows0_v.at[pl.ds(0, 16)],
                        acc_sh.at[pl.ds(NS * RPT, 16)])

    plsc.subcore_barrier()

    base = (NCW * wid + jnp.minimum(wid, NCX)) * B

    def _triple(i, carry):
        off0 = base + (3 * i) * B
        off1 = off0 + B
        off2 = off0 + 2 * B
        ixs = []
        for off, gi_v, di_v, sem in ((off0, gi0_v, di0_v, sem0),
                                     (off1, gi1_v, di1_v, sem1),
                                     (off2, gi2_v, di2_v, sem2)):
            ixs.append((pltpu.async_copy(gidx_hbm.at[pl.ds(off, B)], gi_v,
                                         sem),
                        pltpu.async_copy(dst_hbm.at[pl.ds(off, B)], di_v,
                                         sem)))
        gs = []
        for (ia, ib), gi_v, rows_v, sem in zip(
                ixs, (gi0_v, gi1_v, gi2_v), (rows0_v, rows1_v, rows2_v),
                (sem0, sem1, sem2)):
            ia.wait()
            ib.wait()
            gs.append(pltpu.async_copy(scaled_hbm.at[gi_v], rows_v, sem))
        for g, rows_v, di_v in zip(gs, (rows0_v, rows1_v, rows2_v),
                                   (di0_v, di1_v, di2_v)):
            g.wait()
            pltpu.sync_copy(rows_v, acc_sh.at[di_v], add=True)
        return carry

    lax.fori_loop(0, NIH, _triple, 0)

    @pl.when(wid < NCX)
    def _extra_chunk():
        off = base + NCW * B
        pltpu.sync_copy(gidx_hbm.at[pl.ds(off, B)], gi0_v)
        pltpu.sync_copy(dst_hbm.at[pl.ds(off, B)], di0_v)
        pltpu.async_copy(scaled_hbm.at[gi0_v], rows0_v, sem0).wait()
        pltpu.sync_copy(rows0_v, acc_sh.at[di0_v], add=True)

    plsc.subcore_barrier()
    pltpu.sync_copy(acc_sh.at[pl.ds(r0, RPT)],
                    out_hbm.at[cid, pl.ds(r0, RPT)])

    @pl.when(sid == 0)
    def _flush_tail():
        pltpu.sync_copy(acc_sh.at[pl.ds(NS * RPT, 16)],
                        out_hbm.at[cid, pl.ds(NS * RPT, 16)])


def kernel(x, edge_index, edge_weights, W0, b0, hop1, W1, b1,
           hop2, W2, b2, Wh1, bh1, Wh2, bh2):
    src = edge_index[0]
    dst = edge_index[1]
    gidx = _gidx(src.reshape(NCHUNK, B),
                 edge_weights.reshape(NCHUNK, B)).reshape(E)

    h0, s1 = _mlp_scale(x, W0, b0.reshape(1, D), hop1.reshape(1, K))
    p1 = _sc_agg(s1.reshape(K * N, D), gidx, dst)
    h1, s2 = _combine_scale(h0, p1, W1, b1.reshape(1, D),
                            hop2.reshape(1, K))
    p2 = _sc_agg(s2.reshape(K * N, D), gidx, dst)
    out = _combine_head(h1, p2, W2, b2.reshape(1, D),
                        Wh1, bh1.reshape(1, D), Wh2, bh2.reshape(1, C))
    return out
